# baseline (speedup 1.0000x reference)
import jax
import jax.numpy as jnp
from jax.experimental import pallas as pl
from jax.experimental.pallas import tpu as pltpu

_N = 100000
_D = 512
_H = 128
_K = 8
_TILE = 5000
_GRID = _N // _TILE


def _dot_t(a, b):
    return jax.lax.dot_general(a, b, (((1,), (1,)), ((), ())),
                               preferred_element_type=jnp.float32)


def _select8(vals, logs, largest):
    iota = jax.lax.broadcasted_iota(jnp.int32, vals.shape, 1)
    fill = -jnp.inf if largest else jnp.inf
    out_v = []
    out_l = []
    v = vals
    best = None
    for _ in range(_K):
        best = jnp.max(v) if largest else jnp.min(v)
        idx = jnp.min(jnp.where(v == best, iota, jnp.int32(2 ** 30)))
        onehot = iota == idx
        out_v.append(jnp.reshape(best, (1, 1)))
        out_l.append(jnp.sum(jnp.where(onehot, logs, 0.0), axis=1, keepdims=True))
        v = jnp.where(onehot, fill, v)
    return jnp.concatenate(out_v, axis=1), jnp.concatenate(out_l, axis=1), best


def _fused_kernel(feat_a_ref, feat_b_ref, feat_c_ref, feat_d_ref,
                  W_fc_ref, b_fc_ref, W_a1_ref, b_a1_ref,
                  W_a2_ref, b_a2_ref, W_ic_ref, b_ic_ref, label_ref,
                  out_m_ref, out_loss_ref,
                  m_ref, z_ref, s_ref, tv_ref, tl_ref, bv_ref, bl_ref,
                  thr_t_ref, thr_b_ref):
    i = pl.program_id(0)

    @pl.when(i == 0)
    def _init():
        m_ref[...] = jnp.full_like(m_ref, -jnp.inf)
        z_ref[...] = jnp.zeros_like(z_ref)
        s_ref[...] = jnp.zeros_like(s_ref)
        tv_ref[...] = jnp.full_like(tv_ref, -jnp.inf)
        bv_ref[...] = jnp.full_like(bv_ref, jnp.inf)
        tl_ref[...] = jnp.zeros_like(tl_ref)
        bl_ref[...] = jnp.zeros_like(bl_ref)
        thr_t_ref[0] = -jnp.inf
        thr_b_ref[0] = jnp.inf

    q = _D // 4
    h = (_dot_t(feat_a_ref[...], W_fc_ref[:, 0 * q:1 * q]) +
         _dot_t(feat_b_ref[...], W_fc_ref[:, 1 * q:2 * q]) +
         _dot_t(feat_c_ref[...], W_fc_ref[:, 2 * q:3 * q]) +
         _dot_t(feat_d_ref[...], W_fc_ref[:, 3 * q:4 * q]) + b_fc_ref[...])
    t = jnp.tanh(_dot_t(h, W_a1_ref[...]) + b_a1_ref[...])
    a_row = _dot_t(W_a2_ref[...], t) + b_a2_ref[...]

    tile_max = jnp.max(a_row)
    tile_min = jnp.min(a_row)

    m_old = m_ref[...]
    m_new = jnp.maximum(m_old, tile_max)
    c = jnp.exp(m_old - m_new)
    w = jnp.exp(a_row - m_new)
    m_ref[...] = m_new
    z_ref[...] = z_ref[...] * c + jnp.sum(w)
    s_ref[...] = s_ref[...] * c + jax.lax.dot_general(
        w, h, (((1,), (0,)), ((), ())), preferred_element_type=jnp.float32)

    @pl.when(tile_max > thr_t_ref[0])
    def _merge_top():
        l4 = _dot_t(W_ic_ref[...], h) + b_ic_ref[...]
        cand_v = jnp.concatenate([tv_ref[...], a_row], axis=1)
        cand_l = jnp.concatenate([tl_ref[...], l4], axis=1)
        ntv, ntl, thr = _select8(cand_v, cand_l, largest=True)
        tv_ref[...] = ntv
        tl_ref[...] = ntl
        thr_t_ref[0] = thr

    @pl.when(tile_min < thr_b_ref[0])
    def _merge_bot():
        l4 = _dot_t(W_ic_ref[...], h) + b_ic_ref[...]
        cand_v = jnp.concatenate([bv_ref[...], a_row], axis=1)
        cand_l = jnp.concatenate([bl_ref[...], l4], axis=1)
        nbv, nbl, thr = _select8(cand_v, cand_l, largest=False)
        bv_ref[...] = nbv
        bl_ref[...] = nbl
        thr_b_ref[0] = thr

    @pl.when(i == _GRID - 1)
    def _finalize():
        out_m_ref[...] = s_ref[...] / z_ref[...]
        tl = tl_ref[...]
        bl = bl_ref[...]
        lab = label_ref[...]
        total = jnp.zeros((1, 1), jnp.float32)
        for cls in range(2):
            t0 = tl[2 * cls:2 * cls + 1, :]
            t1 = tl[2 * cls + 1:2 * cls + 2, :]
            mx = jnp.maximum(t0, t1)
            lse_t = mx + jnp.log(jnp.exp(t0 - mx) + jnp.exp(t1 - mx))
            b0 = bl[2 * cls:2 * cls + 1, :]
            b1 = bl[2 * cls + 1:2 * cls + 2, :]
            mxb = jnp.maximum(b0, b1)
            lse_b = mxb + jnp.log(jnp.exp(b0 - mxb) + jnp.exp(b1 - mxb))
            inst = (jnp.sum(lse_t - t1) + jnp.sum(lse_b - b0)) / (2 * _K)
            total = total + jnp.where(lab[0:1, cls:cls + 1] == 1, inst, 0.0)
        out_loss_ref[...] = total


def kernel(features, label, W_fc, b_fc, W_a1, b_a1, W_a2, b_a2, W_ic, b_ic):
    W_ic4 = W_ic.reshape(2 * 2, _H)
    b_ic4 = b_ic.reshape(2 * 2, 1)
    full = lambda shape: pl.BlockSpec(shape, lambda i: (0, 0))
    M, loss = pl.pallas_call(
        _fused_kernel,
        grid=(_GRID,),
        in_specs=[
            pl.BlockSpec((_TILE, _D // 4), lambda i: (i, 0)),
            pl.BlockSpec((_TILE, _D // 4), lambda i: (i, 1)),
            pl.BlockSpec((_TILE, _D // 4), lambda i: (i, 2)),
            pl.BlockSpec((_TILE, _D // 4), lambda i: (i, 3)),
            full((_H, _D)),
            full((1, _H)),
            full((_H, _H)),
            full((1, _H)),
            full((1, _H)),
            full((1, 1)),
            full((4, _H)),
            full((4, 1)),
            full((1, 2)),
        ],
        out_specs=[full((1, _H)), full((1, 1))],
        out_shape=[
            jax.ShapeDtypeStruct((1, _H), jnp.float32),
            jax.ShapeDtypeStruct((1, 1), jnp.float32),
        ],
        scratch_shapes=[
            pltpu.VMEM((1, 1), jnp.float32),
            pltpu.VMEM((1, 1), jnp.float32),
            pltpu.VMEM((1, _H), jnp.float32),
            pltpu.VMEM((1, _K), jnp.float32),
            pltpu.VMEM((4, _K), jnp.float32),
            pltpu.VMEM((1, _K), jnp.float32),
            pltpu.VMEM((4, _K), jnp.float32),
            pltpu.SMEM((1,), jnp.float32),
            pltpu.SMEM((1,), jnp.float32),
        ],
    )(features, features, features, features,
      W_fc, b_fc.reshape(1, _H), W_a1, b_a1.reshape(1, _H),
      W_a2, b_a2.reshape(1, 1), W_ic4, b_ic4, label.reshape(1, 2))
    return (M, loss[0, 0])


# speedup vs baseline: 1.0898x; 1.0898x over previous
import jax
import jax.numpy as jnp
from jax.experimental import pallas as pl
from jax.experimental.pallas import tpu as pltpu

_N = 100000
_D = 512
_H = 128
_K = 8
_TILE = 4000
_GRID = _N // _TILE


def _dot_t(a, b):
    return jax.lax.dot_general(a, b, (((1,), (1,)), ((), ())),
                               preferred_element_type=jnp.float32)


def _select8(vals, logs, largest):
    iota = jax.lax.broadcasted_iota(jnp.int32, vals.shape, 1)
    fill = -jnp.inf if largest else jnp.inf
    out_v = []
    out_l = []
    v = vals
    best = None
    for _ in range(_K):
        best = jnp.max(v) if largest else jnp.min(v)
        idx = jnp.min(jnp.where(v == best, iota, jnp.int32(2 ** 30)))
        onehot = iota == idx
        out_v.append(jnp.reshape(best, (1, 1)))
        out_l.append(jnp.sum(jnp.where(onehot, logs, 0.0), axis=1, keepdims=True))
        v = jnp.where(onehot, fill, v)
    return jnp.concatenate(out_v, axis=1), jnp.concatenate(out_l, axis=1), best


def _fused_kernel(feat_a_ref, feat_b_ref,
                  W_fc_ref, b_fc_ref, W_a1_ref, b_a1_ref,
                  W_a2_ref, b_a2_ref, W_ic_ref, b_ic_ref, label_ref,
                  out_m_ref, out_loss_ref,
                  m_ref, z_ref, s_ref, tv_ref, tl_ref, bv_ref, bl_ref,
                  thr_t_ref, thr_b_ref):
    i = pl.program_id(0)

    @pl.when(i == 0)
    def _init():
        m_ref[...] = jnp.full_like(m_ref, -jnp.inf)
        z_ref[...] = jnp.zeros_like(z_ref)
        s_ref[...] = jnp.zeros_like(s_ref)
        tv_ref[...] = jnp.full_like(tv_ref, -jnp.inf)
        bv_ref[...] = jnp.full_like(bv_ref, jnp.inf)
        tl_ref[...] = jnp.zeros_like(tl_ref)
        bl_ref[...] = jnp.zeros_like(bl_ref)
        thr_t_ref[0] = -jnp.inf
        thr_b_ref[0] = jnp.inf

    feats = jnp.concatenate([feat_a_ref[...], feat_b_ref[...]], axis=0)
    h = _dot_t(feats, W_fc_ref[...]) + b_fc_ref[...]
    t = jnp.tanh(_dot_t(h, W_a1_ref[...]) + b_a1_ref[...])
    a_row = _dot_t(W_a2_ref[...], t) + b_a2_ref[...]

    tile_max = jnp.max(a_row)
    tile_min = jnp.min(a_row)

    m_old = m_ref[...]
    m_new = jnp.maximum(m_old, tile_max)
    c = jnp.exp(m_old - m_new)
    w = jnp.exp(a_row - m_new)
    m_ref[...] = m_new
    z_ref[...] = z_ref[...] * c + jnp.sum(w)
    s_ref[...] = s_ref[...] * c + jax.lax.dot_general(
        w, h, (((1,), (0,)), ((), ())), preferred_element_type=jnp.float32)

    @pl.when(tile_max > thr_t_ref[0])
    def _merge_top():
        l4 = _dot_t(W_ic_ref[...], h) + b_ic_ref[...]
        cand_v = jnp.concatenate([tv_ref[...], a_row], axis=1)
        cand_l = jnp.concatenate([tl_ref[...], l4], axis=1)
        ntv, ntl, thr = _select8(cand_v, cand_l, largest=True)
        tv_ref[...] = ntv
        tl_ref[...] = ntl
        thr_t_ref[0] = thr

    @pl.when(tile_min < thr_b_ref[0])
    def _merge_bot():
        l4 = _dot_t(W_ic_ref[...], h) + b_ic_ref[...]
        cand_v = jnp.concatenate([bv_ref[...], a_row], axis=1)
        cand_l = jnp.concatenate([bl_ref[...], l4], axis=1)
        nbv, nbl, thr = _select8(cand_v, cand_l, largest=False)
        bv_ref[...] = nbv
        bl_ref[...] = nbl
        thr_b_ref[0] = thr

    @pl.when(i == _GRID - 1)
    def _finalize():
        out_m_ref[...] = s_ref[...] / z_ref[...]
        tl = tl_ref[...]
        bl = bl_ref[...]
        lab = label_ref[...]
        total = jnp.zeros((1, 1), jnp.float32)
        for cls in range(2):
            t0 = tl[2 * cls:2 * cls + 1, :]
            t1 = tl[2 * cls + 1:2 * cls + 2, :]
            mx = jnp.maximum(t0, t1)
            lse_t = mx + jnp.log(jnp.exp(t0 - mx) + jnp.exp(t1 - mx))
            b0 = bl[2 * cls:2 * cls + 1, :]
            b1 = bl[2 * cls + 1:2 * cls + 2, :]
            mxb = jnp.maximum(b0, b1)
            lse_b = mxb + jnp.log(jnp.exp(b0 - mxb) + jnp.exp(b1 - mxb))
            inst = (jnp.sum(lse_t - t1) + jnp.sum(lse_b - b0)) / (2 * _K)
            total = total + jnp.where(lab[0:1, cls:cls + 1] == 1, inst, 0.0)
        out_loss_ref[...] = total


def kernel(features, label, W_fc, b_fc, W_a1, b_a1, W_a2, b_a2, W_ic, b_ic):
    W_ic4 = W_ic.reshape(2 * 2, _H)
    b_ic4 = b_ic.reshape(2 * 2, 1)
    full = lambda shape: pl.BlockSpec(shape, lambda i: (0, 0))
    M, loss = pl.pallas_call(
        _fused_kernel,
        grid=(_GRID,),
        in_specs=[
            pl.BlockSpec((_TILE // 2, _D), lambda i: (2 * i, 0)),
            pl.BlockSpec((_TILE // 2, _D), lambda i: (2 * i + 1, 0)),
            full((_H, _D)),
            full((1, _H)),
            full((_H, _H)),
            full((1, _H)),
            full((1, _H)),
            full((1, 1)),
            full((4, _H)),
            full((4, 1)),
            full((1, 2)),
        ],
        out_specs=[full((1, _H)), full((1, 1))],
        out_shape=[
            jax.ShapeDtypeStruct((1, _H), jnp.float32),
            jax.ShapeDtypeStruct((1, 1), jnp.float32),
        ],
        scratch_shapes=[
            pltpu.VMEM((1, 1), jnp.float32),
            pltpu.VMEM((1, 1), jnp.float32),
            pltpu.VMEM((1, _H), jnp.float32),
            pltpu.VMEM((1, _K), jnp.float32),
            pltpu.VMEM((4, _K), jnp.float32),
            pltpu.VMEM((1, _K), jnp.float32),
            pltpu.VMEM((4, _K), jnp.float32),
            pltpu.SMEM((1,), jnp.float32),
            pltpu.SMEM((1,), jnp.float32),
        ],
    )(features, features,
      W_fc, b_fc.reshape(1, _H), W_a1, b_a1.reshape(1, _H),
      W_a2, b_a2.reshape(1, 1), W_ic4, b_ic4, label.reshape(1, 2))
    return (M, loss[0, 0])


# speedup vs baseline: 1.4439x; 1.3249x over previous
import jax
import jax.numpy as jnp
from jax.experimental import pallas as pl
from jax.experimental.pallas import tpu as pltpu

_N = 100000
_D = 512
_H = 128
_K = 8
_TILE = 10000
_GRID = _N // _TILE


def _dot_t(a, b):
    return jax.lax.dot_general(a, b, (((1,), (1,)), ((), ())),
                               preferred_element_type=jnp.float32)


def _select8(vals, logs, largest):
    iota = jax.lax.broadcasted_iota(jnp.int32, vals.shape, 1)
    fill = -jnp.inf if largest else jnp.inf
    out_v = []
    out_l = []
    v = vals
    best = None
    for _ in range(_K):
        best = jnp.max(v) if largest else jnp.min(v)
        idx = jnp.min(jnp.where(v == best, iota, jnp.int32(2 ** 30)))
        onehot = iota == idx
        out_v.append(jnp.reshape(best, (1, 1)))
        out_l.append(jnp.sum(jnp.where(onehot, logs, 0.0), axis=1, keepdims=True))
        v = jnp.where(onehot, fill, v)
    return jnp.concatenate(out_v, axis=1), jnp.concatenate(out_l, axis=1), best


def _fused_kernel(feat_a_ref, feat_b_ref,
                  W_fc_ref, b_fc_ref, W_a1_ref, b_a1_ref,
                  W_a2_ref, b_a2_ref, W_ic_ref, b_ic_ref, label_ref,
                  out_m_ref, out_loss_ref,
                  m_ref, z_ref, s_ref, tv_ref, tl_ref, bv_ref, bl_ref,
                  thr_t_ref, thr_b_ref):
    i = pl.program_id(0)

    @pl.when(i == 0)
    def _init():
        m_ref[...] = jnp.full_like(m_ref, -jnp.inf)
        z_ref[...] = jnp.zeros_like(z_ref)
        s_ref[...] = jnp.zeros_like(s_ref)
        tv_ref[...] = jnp.full_like(tv_ref, -jnp.inf)
        bv_ref[...] = jnp.full_like(bv_ref, jnp.inf)
        tl_ref[...] = jnp.zeros_like(tl_ref)
        bl_ref[...] = jnp.zeros_like(bl_ref)
        thr_t_ref[0] = -jnp.inf
        thr_b_ref[0] = jnp.inf

    h = (_dot_t(feat_a_ref[...], W_fc_ref[:, :_D // 2]) +
         _dot_t(feat_b_ref[...], W_fc_ref[:, _D // 2:]) + b_fc_ref[...])
    t = jnp.tanh(_dot_t(h, W_a1_ref[...]) + b_a1_ref[...])
    a_row = _dot_t(W_a2_ref[...], t) + b_a2_ref[...]

    tile_max = jnp.max(a_row)
    tile_min = jnp.min(a_row)

    m_old = m_ref[...]
    m_new = jnp.maximum(m_old, tile_max)
    c = jnp.exp(m_old - m_new)
    w = jnp.exp(a_row - m_new)
    m_ref[...] = m_new
    z_ref[...] = z_ref[...] * c + jnp.sum(w)
    s_ref[...] = s_ref[...] * c + jax.lax.dot_general(
        w, h, (((1,), (0,)), ((), ())), preferred_element_type=jnp.float32)

    @pl.when(tile_max > thr_t_ref[0])
    def _merge_top():
        l4 = _dot_t(W_ic_ref[...], h) + b_ic_ref[...]
        cand_v = jnp.concatenate([tv_ref[...], a_row], axis=1)
        cand_l = jnp.concatenate([tl_ref[...], l4], axis=1)
        ntv, ntl, thr = _select8(cand_v, cand_l, largest=True)
        tv_ref[...] = ntv
        tl_ref[...] = ntl
        thr_t_ref[0] = thr

    @pl.when(tile_min < thr_b_ref[0])
    def _merge_bot():
        l4 = _dot_t(W_ic_ref[...], h) + b_ic_ref[...]
        cand_v = jnp.concatenate([bv_ref[...], a_row], axis=1)
        cand_l = jnp.concatenate([bl_ref[...], l4], axis=1)
        nbv, nbl, thr = _select8(cand_v, cand_l, largest=False)
        bv_ref[...] = nbv
        bl_ref[...] = nbl
        thr_b_ref[0] = thr

    @pl.when(i == _GRID - 1)
    def _finalize():
        out_m_ref[...] = s_ref[...] / z_ref[...]
        tl = tl_ref[...]
        bl = bl_ref[...]
        lab = label_ref[...]
        total = jnp.zeros((1, 1), jnp.float32)
        for cls in range(2):
            t0 = tl[2 * cls:2 * cls + 1, :]
            t1 = tl[2 * cls + 1:2 * cls + 2, :]
            mx = jnp.maximum(t0, t1)
            lse_t = mx + jnp.log(jnp.exp(t0 - mx) + jnp.exp(t1 - mx))
            b0 = bl[2 * cls:2 * cls + 1, :]
            b1 = bl[2 * cls + 1:2 * cls + 2, :]
            mxb = jnp.maximum(b0, b1)
            lse_b = mxb + jnp.log(jnp.exp(b0 - mxb) + jnp.exp(b1 - mxb))
            inst = (jnp.sum(lse_t - t1) + jnp.sum(lse_b - b0)) / (2 * _K)
            total = total + jnp.where(lab[0:1, cls:cls + 1] == 1, inst, 0.0)
        out_loss_ref[...] = total


def kernel(features, label, W_fc, b_fc, W_a1, b_a1, W_a2, b_a2, W_ic, b_ic):
    W_ic4 = W_ic.reshape(2 * 2, _H)
    b_ic4 = b_ic.reshape(2 * 2, 1)
    full = lambda shape: pl.BlockSpec(shape, lambda i: (0, 0))
    M, loss = pl.pallas_call(
        _fused_kernel,
        grid=(_GRID,),
        in_specs=[
            pl.BlockSpec((_TILE, _D // 2), lambda i: (i, 0)),
            pl.BlockSpec((_TILE, _D // 2), lambda i: (i, 1)),
            full((_H, _D)),
            full((1, _H)),
            full((_H, _H)),
            full((1, _H)),
            full((1, _H)),
            full((1, 1)),
            full((4, _H)),
            full((4, 1)),
            full((1, 2)),
        ],
        out_specs=[full((1, _H)), full((1, 1))],
        out_shape=[
            jax.ShapeDtypeStruct((1, _H), jnp.float32),
            jax.ShapeDtypeStruct((1, 1), jnp.float32),
        ],
        scratch_shapes=[
            pltpu.VMEM((1, 1), jnp.float32),
            pltpu.VMEM((1, 1), jnp.float32),
            pltpu.VMEM((1, _H), jnp.float32),
            pltpu.VMEM((1, _K), jnp.float32),
            pltpu.VMEM((4, _K), jnp.float32),
            pltpu.VMEM((1, _K), jnp.float32),
            pltpu.VMEM((4, _K), jnp.float32),
            pltpu.SMEM((1,), jnp.float32),
            pltpu.SMEM((1,), jnp.float32),
        ],
    )(features, features,
      W_fc, b_fc.reshape(1, _H), W_a1, b_a1.reshape(1, _H),
      W_a2, b_a2.reshape(1, 1), W_ic4, b_ic4, label.reshape(1, 2))
    return (M, loss[0, 0])
